# Initial kernel scaffold; baseline (speedup 1.0000x reference)
import jax
import jax.numpy as jnp
from jax.experimental import pallas as pl


def kernel(actions, params, values, categories, days_since_prev, seasons, action_table, param_table, value_table, category_table, season_table, days_w, days_b, comb_W, comb_b, ln_g, ln_b):
    raise NotImplementedError("write your pallas kernel here")


# speedup vs baseline: 1.9919x; 1.9919x over previous
import functools

import jax
import jax.numpy as jnp
from jax import lax
from jax.experimental import pallas as pl
from jax.experimental.pallas import tpu as pltpu
from jax.experimental.pallas import tpu_sc as plsc

D = 64
NW = 32
CHUNK = 128
ROW_BLOCK = 8192
LN_BLOCK = 2048


def _proj3_body(t0, t1, t2, w0, w1, w2, o0, o1, o2):
    dn = (((1,), (1,)), ((), ()))
    o0[...] = lax.dot_general(t0[...], w0[...], dn, preferred_element_type=jnp.float32)
    o1[...] = lax.dot_general(t1[...], w1[...], dn, preferred_element_type=jnp.float32)
    o2[...] = lax.dot_general(t2[...], w2[...], dn, preferred_element_type=jnp.float32)


def _project_big(tables, ws):
    n = tables[0].shape[0]
    grid = (n + ROW_BLOCK - 1) // ROW_BLOCK
    tspec = pl.BlockSpec((ROW_BLOCK, D), lambda i: (i, 0))
    wspec = pl.BlockSpec((D, D), lambda i: (0, 0))
    return pl.pallas_call(
        _proj3_body,
        grid=(grid,),
        in_specs=[tspec, tspec, tspec, wspec, wspec, wspec],
        out_specs=[tspec, tspec, tspec],
        out_shape=[jax.ShapeDtypeStruct((n, D), jnp.float32)] * 3,
    )(*tables, *ws)


def _proj_small_body(cat, sea, wc, ws, wd, dw, db, cb, oc, os_, ou, ocst):
    dn = (((1,), (1,)), ((), ()))
    oc[...] = lax.dot_general(cat[...], wc[...], dn, preferred_element_type=jnp.float32)
    os_[...] = lax.dot_general(sea[...], ws[...], dn, preferred_element_type=jnp.float32)
    ou[...] = lax.dot_general(dw[...], wd[...], dn, preferred_element_type=jnp.float32)
    ocst[...] = lax.dot_general(db[...], wd[...], dn, preferred_element_type=jnp.float32) + cb[...]


def _project_small(cat, sea, wc, ws, wd, days_w, days_b, comb_b):
    nc, ns = cat.shape[0], sea.shape[0]
    return pl.pallas_call(
        _proj_small_body,
        out_shape=[
            jax.ShapeDtypeStruct((nc, D), jnp.float32),
            jax.ShapeDtypeStruct((ns, D), jnp.float32),
            jax.ShapeDtypeStruct((1, D), jnp.float32),
            jax.ShapeDtypeStruct((1, D), jnp.float32),
        ],
    )(cat, sea, wc, ws, wd, days_w.reshape(1, D), days_b.reshape(1, D),
      comb_b.reshape(1, D))


def _sc_body(ntok, pa, pp, pv, pc, ps, ia, ip, iv, ic, isea, out,
             ia_v, ip_v, iv_v, ic_v, is_v, ba, bp, bv, bc, bs, sem):
    wid = lax.axis_index("s") * 2 + lax.axis_index("c")
    tok_per_w = ntok // NW
    nchunk = tok_per_w // CHUNK

    def chunk_body(j, carry):
        base = wid * tok_per_w + j * CHUNK
        sl = pl.ds(base, CHUNK)
        pltpu.sync_copy(ia.at[sl], ia_v)
        pltpu.sync_copy(ip.at[sl], ip_v)
        pltpu.sync_copy(iv.at[sl], iv_v)
        pltpu.sync_copy(ic.at[sl], ic_v)
        pltpu.sync_copy(isea.at[sl], is_v)
        cps = [
            pltpu.async_copy(pa.at[ia_v], ba, sem),
            pltpu.async_copy(pp.at[ip_v], bp, sem),
            pltpu.async_copy(pv.at[iv_v], bv, sem),
            pltpu.async_copy(pc.at[ic_v], bc, sem),
            pltpu.async_copy(ps.at[is_v], bs, sem),
        ]
        for cp in cps:
            cp.wait()

        def row_body(r, c2):
            for q in range(D // 16):
                s = pl.ds(q * 16, 16)
                ba[r, s] = ba[r, s] + bp[r, s] + bv[r, s] + bc[r, s] + bs[r, s]
            return c2

        lax.fori_loop(0, CHUNK, row_body, 0, unroll=False)
        pltpu.sync_copy(ba, out.at[sl])
        return carry

    lax.fori_loop(0, nchunk, chunk_body, 0, unroll=False)


def _sc_gather_sum(ntok, pa, pp, pv, pc, ps, ia, ip, iv, ic, isea):
    mesh = plsc.VectorSubcoreMesh(core_axis_name="c", subcore_axis_name="s",
                                  num_cores=2, num_subcores=16)
    return pl.kernel(
        functools.partial(_sc_body, ntok),
        out_type=jax.ShapeDtypeStruct((ntok, D), jnp.float32),
        mesh=mesh,
        compiler_params=pltpu.CompilerParams(use_tc_tiling_on_sc=False),
        scratch_types=[
            pltpu.VMEM((CHUNK,), jnp.int32),
            pltpu.VMEM((CHUNK,), jnp.int32),
            pltpu.VMEM((CHUNK,), jnp.int32),
            pltpu.VMEM((CHUNK,), jnp.int32),
            pltpu.VMEM((CHUNK,), jnp.int32),
            pltpu.VMEM((CHUNK, D), jnp.float32),
            pltpu.VMEM((CHUNK, D), jnp.float32),
            pltpu.VMEM((CHUNK, D), jnp.float32),
            pltpu.VMEM((CHUNK, D), jnp.float32),
            pltpu.VMEM((CHUNK, D), jnp.float32),
            pltpu.SemaphoreType.DMA,
        ],
    )(pa, pp, pv, pc, ps, ia, ip, iv, ic, isea)


def _ln_body(s_ref, d_ref, u_ref, c_ref, g_ref, b_ref, o_ref):
    x = s_ref[...] + d_ref[...] * u_ref[...] + c_ref[...]
    mu = jnp.mean(x, axis=1, keepdims=True)
    xc = x - mu
    var = jnp.mean(xc * xc, axis=1, keepdims=True)
    y = xc * lax.rsqrt(var + 1e-5)
    o_ref[...] = y * g_ref[...] + b_ref[...]


def _ln(s, days_col, u, cvec, ln_g, ln_b):
    ntok = s.shape[0]
    grid = ntok // LN_BLOCK
    vspec = pl.BlockSpec((1, D), lambda i: (0, 0))
    return pl.pallas_call(
        _ln_body,
        grid=(grid,),
        in_specs=[
            pl.BlockSpec((LN_BLOCK, D), lambda i: (i, 0)),
            pl.BlockSpec((LN_BLOCK, 1), lambda i: (i, 0)),
            vspec, vspec, vspec, vspec,
        ],
        out_specs=pl.BlockSpec((LN_BLOCK, D), lambda i: (i, 0)),
        out_shape=jax.ShapeDtypeStruct((ntok, D), jnp.float32),
    )(s, days_col, u, cvec, ln_g.reshape(1, D), ln_b.reshape(1, D))


def kernel(actions, params, values, categories, days_since_prev, seasons,
           action_table, param_table, value_table, category_table, season_table,
           days_w, days_b, comb_W, comb_b, ln_g, ln_b):
    B, L = actions.shape
    ntok = B * L

    ia = actions.reshape(ntok).astype(jnp.int32)
    ip = params.reshape(ntok).astype(jnp.int32)
    iv = values.reshape(ntok).astype(jnp.int32)
    ic = categories.reshape(ntok).astype(jnp.int32)
    isea = seasons.reshape(ntok).astype(jnp.int32)
    days_col = days_since_prev.reshape(ntok, 1)

    w_a = comb_W[:, 0 * D:1 * D]
    w_p = comb_W[:, 1 * D:2 * D]
    w_v = comb_W[:, 2 * D:3 * D]
    w_c = comb_W[:, 3 * D:4 * D]
    w_d = comb_W[:, 4 * D:5 * D]
    w_s = comb_W[:, 5 * D:6 * D]

    pa, pp, pv = _project_big((action_table, param_table, value_table),
                              (w_a, w_p, w_v))
    pc, psea, u, cvec = _project_small(category_table, season_table,
                                       w_c, w_s, w_d, days_w, days_b, comb_b)

    s = _sc_gather_sum(ntok, pa, pp, pv, pc, psea, ia, ip, iv, ic, isea)
    out = _ln(s, days_col, u, cvec, ln_g, ln_b)
    return out.reshape(B, L, D)
